# Initial kernel scaffold; baseline (speedup 1.0000x reference)
import jax
import jax.numpy as jnp
from jax.experimental import pallas as pl


def kernel(hidden_states, router_w, w_fc, b_fc, w_proj, b_proj):
    raise NotImplementedError("write your pallas kernel here")


# speedup vs baseline: 2.5253x; 2.5253x over previous
import functools

import jax
import jax.numpy as jnp
from jax.experimental import pallas as pl
from jax.experimental.pallas import tpu as pltpu

S = 2048
E = 64
D_MODEL = 768
D_FF = 3072
T = 128
NB = S // T
NT = NB + E - 1


def _route_kernel(x_ref, rw_ref, gx_ref, dest_ref, counts_ref):
    x = x_ref[...]
    rw = rw_ref[...]
    logits = jax.lax.dot_general(
        x, rw, (((1,), (1,)), ((), ())), preferred_element_type=jnp.float32)
    m = jnp.max(logits, axis=1, keepdims=True)
    e_iota = jax.lax.broadcasted_iota(jnp.int32, (S, E), 1)
    expert = jnp.min(jnp.where(logits == m, e_iota, E), axis=1, keepdims=True)

    onehot = (e_iota == expert).astype(jnp.float32)
    counts = jnp.sum(onehot, axis=0, keepdims=True)
    ei = jax.lax.broadcasted_iota(jnp.int32, (E, E), 0)
    ej = jax.lax.broadcasted_iota(jnp.int32, (E, E), 1)
    upper = (ei < ej).astype(jnp.float32)
    offs = jax.lax.dot_general(
        counts, upper, (((1,), (0,)), ((), ())),
        preferred_element_type=jnp.float32)
    si = jax.lax.broadcasted_iota(jnp.int32, (S, S), 0)
    sj = jax.lax.broadcasted_iota(jnp.int32, (S, S), 1)
    lower = (sj <= si).astype(jnp.float32)
    pref = jax.lax.dot_general(
        lower, onehot, (((1,), (0,)), ((), ())),
        preferred_element_type=jnp.float32)
    rank = jnp.sum(pref * onehot, axis=1, keepdims=True) - 1.0
    tok_off = jax.lax.dot_general(
        onehot, offs, (((1,), (1,)), ((), ())),
        preferred_element_type=jnp.float32)
    dest = (tok_off + rank).astype(jnp.int32)

    perm = (dest == jax.lax.broadcasted_iota(jnp.int32, (S, S), 1))
    perm = perm.astype(jnp.bfloat16)
    gx = jax.lax.dot_general(
        perm, x.astype(jnp.bfloat16), (((0,), (0,)), ((), ())),
        preferred_element_type=jnp.float32)
    gx_ref[...] = gx.astype(jnp.bfloat16)
    dest_ref[...] = dest
    c128 = jnp.concatenate(
        [counts.astype(jnp.int32), jnp.zeros((1, 128 - E), jnp.int32)], axis=1)
    counts_ref[...] = jnp.concatenate(
        [c128, jnp.zeros((7, 128), jnp.int32)], axis=0)


def _gmm_kernel(tb_ref, te_ref, off_ref,
                gx_ref, wfc_ref, bfc_ref, wproj_ref, bproj_ref, gy_ref):
    i = pl.program_id(0)
    e = te_ref[i]
    b = tb_ref[i]
    start = off_ref[e]
    end = off_ref[e + 1]
    x = gx_ref[...]
    h = jax.lax.dot_general(
        x, wfc_ref[0], (((1,), (1,)), ((), ())),
        preferred_element_type=jnp.float32)
    h = jax.nn.gelu(h + bfc_ref[0]).astype(jnp.bfloat16)
    y = jax.lax.dot_general(
        h, wproj_ref[0], (((1,), (1,)), ((), ())),
        preferred_element_type=jnp.float32)
    y = y + bproj_ref[0]
    row = b * T + jax.lax.broadcasted_iota(jnp.int32, (T, 1), 0)
    mask = (row >= start) & (row < end)
    gy_ref[...] = jnp.where(mask, y, gy_ref[...])


def _unperm_kernel(dest_ref, gy_ref, out_ref):
    dest = dest_ref[...]
    perm = (dest == jax.lax.broadcasted_iota(jnp.int32, (S, S), 1))
    out_ref[...] = jax.lax.dot_general(
        perm.astype(jnp.float32), gy_ref[...], (((1,), (0,)), ((), ())),
        preferred_element_type=jnp.float32)


@jax.jit
def kernel(hidden_states, router_w, w_fc, b_fc, w_proj, b_proj):
    x = hidden_states.reshape(S, D_MODEL)

    gx, dest, counts_pad = pl.pallas_call(
        _route_kernel,
        out_shape=[
            jax.ShapeDtypeStruct((S, D_MODEL), jnp.bfloat16),
            jax.ShapeDtypeStruct((S, 1), jnp.int32),
            jax.ShapeDtypeStruct((8, 128), jnp.int32),
        ],
    )(x, router_w)

    counts = counts_pad[0, :E]
    offsets = jnp.concatenate(
        [jnp.zeros((1,), jnp.int32), jnp.cumsum(counts, dtype=jnp.int32)])
    b_idx = jnp.arange(NB, dtype=jnp.int32)[:, None]
    start_e = offsets[:-1][None, :]
    end_e = offsets[1:][None, :]
    valid = (end_e > b_idx * T) & (start_e < (b_idx + 1) * T)
    lin = jnp.arange(NB * E, dtype=jnp.int32)
    key = jnp.where(valid.reshape(-1), lin, NB * E)
    order = jnp.sort(key)[:NT]
    is_pad = order >= NB * E
    e_last = jnp.sum((offsets[:E] < S).astype(jnp.int32)) - 1
    tile_b = jnp.where(is_pad, NB - 1, order // E).astype(jnp.int32)
    tile_e = jnp.where(is_pad, e_last, order % E).astype(jnp.int32)

    gy = pl.pallas_call(
        _gmm_kernel,
        grid_spec=pltpu.PrefetchScalarGridSpec(
            num_scalar_prefetch=3,
            grid=(NT,),
            in_specs=[
                pl.BlockSpec((T, D_MODEL), lambda i, tb, te, off: (tb[i], 0)),
                pl.BlockSpec((1, D_FF, D_MODEL),
                             lambda i, tb, te, off: (te[i], 0, 0)),
                pl.BlockSpec((1, 1, D_FF),
                             lambda i, tb, te, off: (te[i], 0, 0)),
                pl.BlockSpec((1, D_MODEL, D_FF),
                             lambda i, tb, te, off: (te[i], 0, 0)),
                pl.BlockSpec((1, 1, D_MODEL),
                             lambda i, tb, te, off: (te[i], 0, 0)),
            ],
            out_specs=pl.BlockSpec((T, D_MODEL), lambda i, tb, te, off: (tb[i], 0)),
        ),
        out_shape=jax.ShapeDtypeStruct((S, D_MODEL), jnp.float32),
    )(tile_b, tile_e, offsets,
      gx,
      w_fc.astype(jnp.bfloat16),
      b_fc.reshape(E, 1, D_FF),
      w_proj.astype(jnp.bfloat16),
      b_proj.reshape(E, 1, D_MODEL))

    out = pl.pallas_call(
        _unperm_kernel,
        out_shape=jax.ShapeDtypeStruct((S, D_MODEL), jnp.float32),
    )(dest, gy)
    return out.reshape(hidden_states.shape)


# speedup vs baseline: 4.6565x; 1.8439x over previous
import functools

import jax
import jax.numpy as jnp
from jax.experimental import pallas as pl
from jax.experimental.pallas import tpu as pltpu

S = 2048
E = 64
D_MODEL = 768
D_FF = 3072
T = 128
NB = S // T
NT = NB + E - 1


def _route_kernel(x_ref, rw_ref, gx_ref, dest_ref, counts_ref):
    x = x_ref[...]
    rw = rw_ref[...]
    logits = jax.lax.dot_general(
        x, rw, (((1,), (1,)), ((), ())), preferred_element_type=jnp.float32)
    m = jnp.max(logits, axis=1, keepdims=True)
    e_iota = jax.lax.broadcasted_iota(jnp.int32, (S, E), 1)
    expert = jnp.min(jnp.where(logits == m, e_iota, E), axis=1, keepdims=True)

    onehot = (e_iota == expert).astype(jnp.float32)
    counts = jnp.sum(onehot, axis=0, keepdims=True)
    ei = jax.lax.broadcasted_iota(jnp.int32, (E, E), 0)
    ej = jax.lax.broadcasted_iota(jnp.int32, (E, E), 1)
    upper = (ei < ej).astype(jnp.float32)
    offs = jax.lax.dot_general(
        counts, upper, (((1,), (0,)), ((), ())),
        preferred_element_type=jnp.float32)
    si = jax.lax.broadcasted_iota(jnp.int32, (S, S), 0)
    sj = jax.lax.broadcasted_iota(jnp.int32, (S, S), 1)
    lower = (sj <= si).astype(jnp.float32)
    pref = jax.lax.dot_general(
        lower, onehot, (((1,), (0,)), ((), ())),
        preferred_element_type=jnp.float32)
    rank = jnp.sum(pref * onehot, axis=1, keepdims=True) - 1.0
    tok_off = jax.lax.dot_general(
        onehot, offs, (((1,), (1,)), ((), ())),
        preferred_element_type=jnp.float32)
    dest = (tok_off + rank).astype(jnp.int32)

    perm = (dest == jax.lax.broadcasted_iota(jnp.int32, (S, S), 1))
    perm = perm.astype(jnp.bfloat16)
    gx = jax.lax.dot_general(
        perm, x.astype(jnp.bfloat16), (((0,), (0,)), ((), ())),
        preferred_element_type=jnp.float32)
    gx_ref[...] = gx.astype(jnp.bfloat16)
    dest_ref[...] = dest
    c128 = jnp.concatenate(
        [counts.astype(jnp.int32), jnp.zeros((1, 128 - E), jnp.int32)], axis=1)
    counts_ref[...] = jnp.concatenate(
        [c128, jnp.zeros((7, 128), jnp.int32)], axis=0)


def _gmm_kernel(tb_ref, te_ref, off_ref,
                gx_ref, wfc_ref, bfc_ref, wproj_ref, bproj_ref, gy_ref):
    i = pl.program_id(0)
    e = te_ref[i]
    b = tb_ref[i]
    start = off_ref[e]
    end = off_ref[e + 1]
    x = gx_ref[...]
    wfc = wfc_ref[0].astype(jnp.bfloat16)
    h = jax.lax.dot_general(
        x, wfc, (((1,), (1,)), ((), ())),
        preferred_element_type=jnp.float32)
    h = jax.nn.gelu(h + bfc_ref[0]).astype(jnp.bfloat16)
    wproj = wproj_ref[0].astype(jnp.bfloat16)
    y = jax.lax.dot_general(
        h, wproj, (((1,), (1,)), ((), ())),
        preferred_element_type=jnp.float32)
    y = y + bproj_ref[0]
    row = b * T + jax.lax.broadcasted_iota(jnp.int32, (T, 1), 0)
    mask = (row >= start) & (row < end)
    gy_ref[...] = jnp.where(mask, y, gy_ref[...])


def _unperm_kernel(dest_ref, gy_ref, out_ref):
    dest = dest_ref[...]
    perm = (dest == jax.lax.broadcasted_iota(jnp.int32, (S, S), 1))
    out_ref[...] = jax.lax.dot_general(
        perm.astype(jnp.float32), gy_ref[...], (((1,), (0,)), ((), ())),
        preferred_element_type=jnp.float32)


@jax.jit
def kernel(hidden_states, router_w, w_fc, b_fc, w_proj, b_proj):
    x = hidden_states.reshape(S, D_MODEL)

    gx, dest, counts_pad = pl.pallas_call(
        _route_kernel,
        out_shape=[
            jax.ShapeDtypeStruct((S, D_MODEL), jnp.bfloat16),
            jax.ShapeDtypeStruct((S, 1), jnp.int32),
            jax.ShapeDtypeStruct((8, 128), jnp.int32),
        ],
    )(x, router_w)

    counts = counts_pad[0, :E]
    offsets = jnp.concatenate(
        [jnp.zeros((1,), jnp.int32), jnp.cumsum(counts, dtype=jnp.int32)])
    b_idx = jnp.arange(NB, dtype=jnp.int32)[:, None]
    start_e = offsets[:-1][None, :]
    end_e = offsets[1:][None, :]
    valid = (end_e > b_idx * T) & (start_e < (b_idx + 1) * T)
    lin = jnp.arange(NB * E, dtype=jnp.int32)
    key = jnp.where(valid.reshape(-1), lin, NB * E)
    order = jnp.sort(key)[:NT]
    is_pad = order >= NB * E
    e_last = jnp.sum((offsets[:E] < S).astype(jnp.int32)) - 1
    tile_b = jnp.where(is_pad, NB - 1, order // E).astype(jnp.int32)
    tile_e = jnp.where(is_pad, e_last, order % E).astype(jnp.int32)

    gy = pl.pallas_call(
        _gmm_kernel,
        grid_spec=pltpu.PrefetchScalarGridSpec(
            num_scalar_prefetch=3,
            grid=(NT,),
            in_specs=[
                pl.BlockSpec((T, D_MODEL), lambda i, tb, te, off: (tb[i], 0)),
                pl.BlockSpec((1, D_FF, D_MODEL),
                             lambda i, tb, te, off: (te[i], 0, 0)),
                pl.BlockSpec((1, 1, D_FF),
                             lambda i, tb, te, off: (te[i], 0, 0)),
                pl.BlockSpec((1, D_MODEL, D_FF),
                             lambda i, tb, te, off: (te[i], 0, 0)),
                pl.BlockSpec((1, 1, D_MODEL),
                             lambda i, tb, te, off: (te[i], 0, 0)),
            ],
            out_specs=pl.BlockSpec((T, D_MODEL), lambda i, tb, te, off: (tb[i], 0)),
        ),
        out_shape=jax.ShapeDtypeStruct((S, D_MODEL), jnp.float32),
    )(tile_b, tile_e, offsets,
      gx,
      w_fc,
      b_fc.reshape(E, 1, D_FF),
      w_proj,
      b_proj.reshape(E, 1, D_MODEL))

    out = pl.pallas_call(
        _unperm_kernel,
        out_shape=jax.ShapeDtypeStruct((S, D_MODEL), jnp.float32),
    )(dest, gy)
    return out.reshape(hidden_states.shape)


# speedup vs baseline: 5.0665x; 1.0881x over previous
import functools

import jax
import jax.numpy as jnp
from jax.experimental import pallas as pl
from jax.experimental.pallas import tpu as pltpu

S = 2048
E = 64
D_MODEL = 768
D_FF = 3072
T = 256
NB = S // T
NT = NB + E - 1


def _route_kernel(x_ref, rw_ref, gx_ref, dest_ref, counts_ref):
    x = x_ref[...]
    rw = rw_ref[...]
    logits = jax.lax.dot_general(
        x, rw, (((1,), (1,)), ((), ())), preferred_element_type=jnp.float32)
    m = jnp.max(logits, axis=1, keepdims=True)
    e_iota = jax.lax.broadcasted_iota(jnp.int32, (S, E), 1)
    expert = jnp.min(jnp.where(logits == m, e_iota, E), axis=1, keepdims=True)

    onehot = (e_iota == expert).astype(jnp.float32)
    counts = jnp.sum(onehot, axis=0, keepdims=True)
    ei = jax.lax.broadcasted_iota(jnp.int32, (E, E), 0)
    ej = jax.lax.broadcasted_iota(jnp.int32, (E, E), 1)
    upper = (ei < ej).astype(jnp.float32)
    offs = jax.lax.dot_general(
        counts, upper, (((1,), (0,)), ((), ())),
        preferred_element_type=jnp.float32)
    si = jax.lax.broadcasted_iota(jnp.int32, (S, S), 0)
    sj = jax.lax.broadcasted_iota(jnp.int32, (S, S), 1)
    lower = (sj <= si).astype(jnp.float32)
    pref = jax.lax.dot_general(
        lower, onehot, (((1,), (0,)), ((), ())),
        preferred_element_type=jnp.float32)
    rank = jnp.sum(pref * onehot, axis=1, keepdims=True) - 1.0
    tok_off = jax.lax.dot_general(
        onehot, offs, (((1,), (1,)), ((), ())),
        preferred_element_type=jnp.float32)
    dest = (tok_off + rank).astype(jnp.int32)

    perm = (dest == jax.lax.broadcasted_iota(jnp.int32, (S, S), 1))
    perm = perm.astype(jnp.bfloat16)
    gx = jax.lax.dot_general(
        perm, x.astype(jnp.bfloat16), (((0,), (0,)), ((), ())),
        preferred_element_type=jnp.float32)
    gx_ref[...] = gx.astype(jnp.bfloat16)
    dest_ref[...] = dest
    c128 = jnp.concatenate(
        [counts.astype(jnp.int32), jnp.zeros((1, 128 - E), jnp.int32)], axis=1)
    counts_ref[...] = jnp.concatenate(
        [c128, jnp.zeros((7, 128), jnp.int32)], axis=0)


def _gmm_kernel(tb_ref, te_ref, off_ref,
                gx_ref, wfc_ref, bfc_ref, wproj_ref, bproj_ref, gy_ref):
    i = pl.program_id(0)
    e = te_ref[i]
    b = tb_ref[i]
    start = off_ref[e]
    end = off_ref[e + 1]
    x = gx_ref[...]
    wfc = wfc_ref[0].astype(jnp.bfloat16)
    h = jax.lax.dot_general(
        x, wfc, (((1,), (1,)), ((), ())),
        preferred_element_type=jnp.float32)
    h = jax.nn.gelu(h + bfc_ref[0]).astype(jnp.bfloat16)
    wproj = wproj_ref[0].astype(jnp.bfloat16)
    y = jax.lax.dot_general(
        h, wproj, (((1,), (1,)), ((), ())),
        preferred_element_type=jnp.float32)
    y = y + bproj_ref[0]
    row = b * T + jax.lax.broadcasted_iota(jnp.int32, (T, 1), 0)
    mask = (row >= start) & (row < end)
    gy_ref[...] = jnp.where(mask, y, gy_ref[...])


def _unperm_kernel(dest_ref, gy_ref, out_ref):
    dest = dest_ref[...]
    perm = (dest == jax.lax.broadcasted_iota(jnp.int32, (S, S), 1))
    out_ref[...] = jax.lax.dot_general(
        perm.astype(jnp.float32), gy_ref[...], (((1,), (0,)), ((), ())),
        preferred_element_type=jnp.float32)


@jax.jit
def kernel(hidden_states, router_w, w_fc, b_fc, w_proj, b_proj):
    x = hidden_states.reshape(S, D_MODEL)

    gx, dest, counts_pad = pl.pallas_call(
        _route_kernel,
        out_shape=[
            jax.ShapeDtypeStruct((S, D_MODEL), jnp.bfloat16),
            jax.ShapeDtypeStruct((S, 1), jnp.int32),
            jax.ShapeDtypeStruct((8, 128), jnp.int32),
        ],
    )(x, router_w)

    counts = counts_pad[0, :E]
    offsets = jnp.concatenate(
        [jnp.zeros((1,), jnp.int32), jnp.cumsum(counts, dtype=jnp.int32)])
    b_idx = jnp.arange(NB, dtype=jnp.int32)[:, None]
    start_e = offsets[:-1][None, :]
    end_e = offsets[1:][None, :]
    valid = (end_e > b_idx * T) & (start_e < (b_idx + 1) * T)
    lin = jnp.arange(NB * E, dtype=jnp.int32)
    key = jnp.where(valid.reshape(-1), lin, NB * E)
    order = jnp.sort(key)[:NT]
    is_pad = order >= NB * E
    e_last = jnp.sum((offsets[:E] < S).astype(jnp.int32)) - 1
    tile_b = jnp.where(is_pad, NB - 1, order // E).astype(jnp.int32)
    tile_e = jnp.where(is_pad, e_last, order % E).astype(jnp.int32)

    gy = pl.pallas_call(
        _gmm_kernel,
        grid_spec=pltpu.PrefetchScalarGridSpec(
            num_scalar_prefetch=3,
            grid=(NT,),
            in_specs=[
                pl.BlockSpec((T, D_MODEL), lambda i, tb, te, off: (tb[i], 0)),
                pl.BlockSpec((1, D_FF, D_MODEL),
                             lambda i, tb, te, off: (te[i], 0, 0)),
                pl.BlockSpec((1, 1, D_FF),
                             lambda i, tb, te, off: (te[i], 0, 0)),
                pl.BlockSpec((1, D_MODEL, D_FF),
                             lambda i, tb, te, off: (te[i], 0, 0)),
                pl.BlockSpec((1, 1, D_MODEL),
                             lambda i, tb, te, off: (te[i], 0, 0)),
            ],
            out_specs=pl.BlockSpec((T, D_MODEL), lambda i, tb, te, off: (tb[i], 0)),
        ),
        out_shape=jax.ShapeDtypeStruct((S, D_MODEL), jnp.float32),
    )(tile_b, tile_e, offsets,
      gx,
      w_fc,
      b_fc.reshape(E, 1, D_FF),
      w_proj,
      b_proj.reshape(E, 1, D_MODEL))

    out = pl.pallas_call(
        _unperm_kernel,
        out_shape=jax.ShapeDtypeStruct((S, D_MODEL), jnp.float32),
    )(dest, gy)
    return out.reshape(hidden_states.shape)
